# baseline (speedup 1.0000x reference)
import functools

import jax
import jax.numpy as jnp
from jax import lax
from jax.experimental import pallas as pl
from jax.experimental.pallas import tpu as pltpu
from jax.experimental.pallas import tpu_sc as plsc

_N = 100000
_D1 = 256
_D2 = 512
_NC, _NS, _L = 2, 16, 16
_NW = _NC * _NS
_CH = 40
_NB = 6
_G1 = _D1 // _L
_G2 = _D2 // _L
_BLK = 8

_S_A = 40000
_S_C = 44000
_TCB = 4000
_GRID_A = (_N - _S_A) // _TCB
_GRID_C = (_N - _S_C) // _TCB
_OFF_A = _S_A // _TCB
_OFF_C = _S_C // _TCB

_mesh = plsc.VectorSubcoreMesh(
    core_axis_name="c", subcore_axis_name="s",
    num_cores=_NC, num_subcores=_NS)


def _worker_tail(wid, nch, s_rows):
    s_main = _NW * _CH * nch
    t = (s_rows - s_main) // 8
    q, r = divmod(t, _NW)
    n_units = q + jnp.where(wid < r, 1, 0)
    tail_base = s_main + 8 * (q * wid + jnp.minimum(wid, r))
    max_rows = 8 * (q + (1 if r else 0))
    return tail_base, 8 * n_units, max_rows


def _make_sc_stats(nch, s_rows):
    rw = _CH * nch

    @functools.partial(
        pl.kernel,
        out_type=jax.ShapeDtypeStruct((_NW, 1, 2 * _D1), jnp.float32),
        mesh=_mesh,
        scratch_types=[
            pltpu.VMEM((_NB, _CH, _D1), jnp.float32),
            pltpu.VMEM((1, 2 * _D1), jnp.float32),
            pltpu.SemaphoreType.DMA((_NB,)),
        ],
    )
    def sc_stats(md_hbm, out_hbm, buf, statbuf, sems):
        wid = lax.axis_index("s") * _NC + lax.axis_index("c")
        base = wid * rw

        def chunk_src(c):
            return md_hbm.at[pl.ds(base + c * _CH, _CH)]

        def make_rows2_body(b):
            def body(i, carry):
                out = list(carry)
                r0 = 2 * i
                for c in range(_G1):
                    v0 = buf[b, r0, pl.ds(c * _L, _L)]
                    v1 = buf[b, r0 + 1, pl.ds(c * _L, _L)]
                    out[c] = (out[c] + v0) + v1
                    out[_G1 + c] = (out[_G1 + c] + v0 * v0) + v1 * v1
                return tuple(out)
            return body

        def make_row_body(b):
            def body(r, carry):
                out = list(carry)
                for c in range(_G1):
                    v = buf[b, r, pl.ds(c * _L, _L)]
                    out[c] = out[c] + v
                    out[_G1 + c] = out[_G1 + c] + v * v
                return tuple(out)
            return body

        for c in range(_NB - 1):
            pltpu.async_copy(chunk_src(c), buf.at[c], sems.at[c])

        def chunk_body(g, carry):
            b = lax.rem(g, _NB)
            pltpu.make_async_copy(chunk_src(g), buf.at[b], sems.at[b]).wait()

            @pl.when(g + _NB - 1 < nch)
            def _():
                b2 = lax.rem(g + _NB - 1, _NB)
                pltpu.async_copy(chunk_src(g + _NB - 1), buf.at[b2],
                                 sems.at[b2])

            return lax.fori_loop(0, _CH // 2, make_rows2_body(b), carry)

        acc = tuple(jnp.zeros((_L,), jnp.float32) for _ in range(2 * _G1))
        acc = lax.fori_loop(0, nch, chunk_body, acc)

        tail_base, tail_rows, max_rows = _worker_tail(wid, nch, s_rows)
        if max_rows:
            @pl.when(tail_rows > 0)
            def _():
                pltpu.sync_copy(md_hbm.at[pl.ds(tail_base, max_rows)],
                                buf.at[0, pl.ds(0, max_rows)])

            acc = lax.fori_loop(0, tail_rows, make_row_body(0), acc)

        for c in range(_G1):
            statbuf[0, pl.ds(c * _L, _L)] = acc[c]
            statbuf[0, pl.ds(_D1 + c * _L, _L)] = acc[_G1 + c]
        pltpu.sync_copy(statbuf, out_hbm.at[wid])

    return sc_stats


_sc_stats = _make_sc_stats(31, _S_A)


def _tc_stats_body(md_ref, sums_ref, sumsq_ref, acc_s, acc_q):
    i = pl.program_id(0)

    @pl.when(i == 0)
    def _():
        acc_s[...] = jnp.zeros_like(acc_s)
        acc_q[...] = jnp.zeros_like(acc_q)

    x = md_ref[...]
    acc_s[...] += jnp.sum(x, axis=0, keepdims=True)
    acc_q[...] += jnp.sum(x * x, axis=0, keepdims=True)

    @pl.when(i == _GRID_A - 1)
    def _():
        sums_ref[...] = acc_s[...]
        sumsq_ref[...] = acc_q[...]


_tc_stats = pl.pallas_call(
    _tc_stats_body,
    grid=(_GRID_A,),
    in_specs=[pl.BlockSpec((_TCB, _D1), lambda i: (i + _OFF_A, 0))],
    out_specs=[pl.BlockSpec((1, _D1), lambda i: (0, 0)),
               pl.BlockSpec((1, _D1), lambda i: (0, 0))],
    out_shape=[jax.ShapeDtypeStruct((1, _D1), jnp.float32),
               jax.ShapeDtypeStruct((1, _D1), jnp.float32)],
    scratch_shapes=[pltpu.VMEM((1, _D1), jnp.float32),
                    pltpu.VMEM((1, _D1), jnp.float32)],
)


def _encoder_body(parts_ref, tsum_ref, tsumsq_ref, x_ref, w_ref, b_ref,
                  out_ref):
    parts = parts_ref[:, 0, :]
    sums = (jnp.sum(parts[:, :_D1], axis=0, keepdims=True) + tsum_ref[...])
    sumsq = (jnp.sum(parts[:, _D1:], axis=0, keepdims=True) + tsumsq_ref[...])
    mean = sums / _N
    var = jnp.maximum((sumsq - sums * mean) / (_N - 1), 0.0)
    std = jnp.sqrt(var)
    new = (x_ref[...] - mean) / std
    new = jnp.where(std == 0.0, jnp.zeros_like(new), new)
    z = jnp.dot(new, w_ref[...], preferred_element_type=jnp.float32)
    out_ref[...] = jnp.tanh(z + b_ref[...])


_encoder = pl.pallas_call(
    _encoder_body,
    out_shape=jax.ShapeDtypeStruct((1, _D2), jnp.float32),
)


def _make_sc_dist(nch, s_rows):
    rw = _CH * nch

    @functools.partial(
        pl.kernel,
        out_type=jax.ShapeDtypeStruct((_NW, 1, _L), jnp.float32),
        mesh=_mesh,
        scratch_types=[
            pltpu.VMEM((_NB, _CH, _D2), jnp.float32),
            pltpu.VMEM((_D2,), jnp.float32),
            pltpu.VMEM((1, _L), jnp.float32),
            pltpu.SemaphoreType.DMA((_NB,)),
        ],
    )
    def sc_dist(mem_hbm, e_hbm, out_hbm, buf, e_v, min_v, sems):
        wid = lax.axis_index("s") * _NC + lax.axis_index("c")
        base = wid * rw

        pltpu.sync_copy(e_hbm.at[0], e_v)

        iota16 = lax.iota(jnp.int32, _L)
        dnums = lax.GatherDimensionNumbers(
            offset_dims=(), collapsed_slice_dims=(0,), start_index_map=(0,))
        perms = [(iota16 ^ k).reshape(_L, 1) for k in (1, 2, 4, 8)]

        def lane_total(v):
            for perm in perms:
                v = v + lax.gather(
                    v, perm, dnums, slice_sizes=(1,),
                    mode=lax.GatherScatterMode.PROMISE_IN_BOUNDS)
            return v

        def make_block_body(b):
            def body(i, m):
                r0 = _BLK * i
                ch = [[None, None] for _ in range(_BLK)]
                for c in range(_G2):
                    ev = e_v[pl.ds(c * _L, _L)]
                    for r in range(_BLK):
                        d = jnp.abs(buf[b, r0 + r, pl.ds(c * _L, _L)] - ev)
                        j = c % 2
                        ch[r][j] = d if ch[r][j] is None else ch[r][j] + d
                for r in range(_BLK):
                    m = jnp.minimum(m, lane_total(ch[r][0] + ch[r][1]))
                return m
            return body

        def make_row_body(b):
            def body(r, m):
                a0 = jnp.abs(buf[b, r, pl.ds(0, _L)] - e_v[pl.ds(0, _L)])
                a1 = jnp.abs(buf[b, r, pl.ds(_L, _L)] - e_v[pl.ds(_L, _L)])
                for c in range(2, _G2):
                    d = jnp.abs(buf[b, r, pl.ds(c * _L, _L)]
                                - e_v[pl.ds(c * _L, _L)])
                    if c % 2 == 0:
                        a0 = a0 + d
                    else:
                        a1 = a1 + d
                return jnp.minimum(m, lane_total(a0 + a1))
            return body

        def chunk_src(c):
            return mem_hbm.at[pl.ds(base + c * _CH, _CH)]

        for c in range(_NB - 1):
            pltpu.async_copy(chunk_src(c), buf.at[c], sems.at[c])

        def chunk_body(g, m):
            b = lax.rem(g, _NB)
            pltpu.make_async_copy(chunk_src(g), buf.at[b], sems.at[b]).wait()

            @pl.when(g + _NB - 1 < nch)
            def _():
                b2 = lax.rem(g + _NB - 1, _NB)
                pltpu.async_copy(chunk_src(g + _NB - 1), buf.at[b2],
                                 sems.at[b2])

            return lax.fori_loop(0, _CH // _BLK, make_block_body(b), m)

        m = jnp.full((_L,), jnp.inf, jnp.float32)
        m = lax.fori_loop(0, nch, chunk_body, m)

        tail_base, tail_rows, max_rows = _worker_tail(wid, nch, s_rows)
        if max_rows:
            @pl.when(tail_rows > 0)
            def _():
                pltpu.sync_copy(mem_hbm.at[pl.ds(tail_base, max_rows)],
                                buf.at[0, pl.ds(0, max_rows)])

            m = lax.fori_loop(0, tail_rows, make_row_body(0), m)

        min_v[...] = m.reshape(1, _L)
        pltpu.sync_copy(min_v, out_hbm.at[wid])

    return sc_dist


_sc_dist = _make_sc_dist(34, _S_C)


def _tc_dist_body(mem_ref, e_ref, out_ref, acc_m):
    i = pl.program_id(0)

    @pl.when(i == 0)
    def _():
        acc_m[...] = jnp.full_like(acc_m, jnp.inf)

    z = jnp.abs(mem_ref[...] - e_ref[...])
    ones = jnp.ones((_D2, 1), jnp.float32)
    d = jnp.dot(z, ones, preferred_element_type=jnp.float32)
    acc_m[...] = jnp.minimum(acc_m[...], jnp.min(d))

    @pl.when(i == _GRID_C - 1)
    def _():
        out_ref[...] = acc_m[...]


_tc_dist = pl.pallas_call(
    _tc_dist_body,
    grid=(_GRID_C,),
    in_specs=[pl.BlockSpec((_TCB, _D2), lambda i: (i + _OFF_C, 0)),
              pl.BlockSpec((1, _D2), lambda i: (0, 0))],
    out_specs=pl.BlockSpec((1, 1), lambda i: (0, 0)),
    out_shape=jax.ShapeDtypeStruct((1, 1), jnp.float32),
    scratch_shapes=[pltpu.VMEM((1, 1), jnp.float32)],
)


def kernel(x, memory, mem_data, W_enc, b_enc):
    parts_sc = _sc_stats(mem_data)
    tsum, tsumsq = _tc_stats(mem_data)
    e = _encoder(parts_sc, tsum, tsumsq, x, W_enc, b_enc.reshape(1, _D2))
    mins_sc = _sc_dist(memory, e)
    min_tc = _tc_dist(memory, e)
    return jnp.minimum(jnp.min(mins_sc), min_tc[0, 0])


# speedup vs baseline: 1.0043x; 1.0043x over previous
import functools

import jax
import jax.numpy as jnp
from jax import lax
from jax.experimental import pallas as pl
from jax.experimental.pallas import tpu as pltpu
from jax.experimental.pallas import tpu_sc as plsc

_N = 100000
_D1 = 256
_D2 = 512
_NC, _NS, _L = 2, 16, 16
_NW = _NC * _NS
_CH = 80
_NB = 3
_G1 = _D1 // _L
_G2 = _D2 // _L
_BLK = 8

_S_A = 40000
_S_C = 44000
_TCB = 4000
_GRID_A = (_N - _S_A) // _TCB
_GRID_C = (_N - _S_C) // _TCB
_OFF_A = _S_A // _TCB
_OFF_C = _S_C // _TCB

_mesh = plsc.VectorSubcoreMesh(
    core_axis_name="c", subcore_axis_name="s",
    num_cores=_NC, num_subcores=_NS)


def _worker_tail(wid, nch, s_rows):
    s_main = _NW * _CH * nch
    t = (s_rows - s_main) // 8
    q, r = divmod(t, _NW)
    n_units = q + jnp.where(wid < r, 1, 0)
    tail_base = s_main + 8 * (q * wid + jnp.minimum(wid, r))
    max_rows = 8 * (q + (1 if r else 0))
    return tail_base, 8 * n_units, max_rows


def _make_sc_stats(nch, s_rows):
    rw = _CH * nch

    @functools.partial(
        pl.kernel,
        out_type=jax.ShapeDtypeStruct((_NW, 1, 2 * _D1), jnp.float32),
        mesh=_mesh,
        scratch_types=[
            pltpu.VMEM((_NB, _CH, _D1), jnp.float32),
            pltpu.VMEM((1, 2 * _D1), jnp.float32),
            pltpu.SemaphoreType.DMA((_NB,)),
        ],
    )
    def sc_stats(md_hbm, out_hbm, buf, statbuf, sems):
        wid = lax.axis_index("s") * _NC + lax.axis_index("c")
        base = wid * rw

        def chunk_src(c):
            return md_hbm.at[pl.ds(base + c * _CH, _CH)]

        def make_rows2_body(b):
            def body(i, carry):
                out = list(carry)
                r0 = 2 * i
                for c in range(_G1):
                    v0 = buf[b, r0, pl.ds(c * _L, _L)]
                    v1 = buf[b, r0 + 1, pl.ds(c * _L, _L)]
                    out[c] = (out[c] + v0) + v1
                    out[_G1 + c] = (out[_G1 + c] + v0 * v0) + v1 * v1
                return tuple(out)
            return body

        def make_row_body(b):
            def body(r, carry):
                out = list(carry)
                for c in range(_G1):
                    v = buf[b, r, pl.ds(c * _L, _L)]
                    out[c] = out[c] + v
                    out[_G1 + c] = out[_G1 + c] + v * v
                return tuple(out)
            return body

        for c in range(_NB - 1):
            pltpu.async_copy(chunk_src(c), buf.at[c], sems.at[c])

        def chunk_body(g, carry):
            b = lax.rem(g, _NB)
            pltpu.make_async_copy(chunk_src(g), buf.at[b], sems.at[b]).wait()

            @pl.when(g + _NB - 1 < nch)
            def _():
                b2 = lax.rem(g + _NB - 1, _NB)
                pltpu.async_copy(chunk_src(g + _NB - 1), buf.at[b2],
                                 sems.at[b2])

            return lax.fori_loop(0, _CH // 2, make_rows2_body(b), carry)

        acc = tuple(jnp.zeros((_L,), jnp.float32) for _ in range(2 * _G1))
        acc = lax.fori_loop(0, nch, chunk_body, acc)

        tail_base, tail_rows, max_rows = _worker_tail(wid, nch, s_rows)
        if max_rows:
            @pl.when(tail_rows > 0)
            def _():
                pltpu.sync_copy(md_hbm.at[pl.ds(tail_base, max_rows)],
                                buf.at[0, pl.ds(0, max_rows)])

            acc = lax.fori_loop(0, tail_rows, make_row_body(0), acc)

        for c in range(_G1):
            statbuf[0, pl.ds(c * _L, _L)] = acc[c]
            statbuf[0, pl.ds(_D1 + c * _L, _L)] = acc[_G1 + c]
        pltpu.sync_copy(statbuf, out_hbm.at[wid])

    return sc_stats


_sc_stats = _make_sc_stats(15, _S_A)


def _tc_stats_body(md_ref, sums_ref, sumsq_ref, acc_s, acc_q):
    i = pl.program_id(0)

    @pl.when(i == 0)
    def _():
        acc_s[...] = jnp.zeros_like(acc_s)
        acc_q[...] = jnp.zeros_like(acc_q)

    x = md_ref[...]
    acc_s[...] += jnp.sum(x, axis=0, keepdims=True)
    acc_q[...] += jnp.sum(x * x, axis=0, keepdims=True)

    @pl.when(i == _GRID_A - 1)
    def _():
        sums_ref[...] = acc_s[...]
        sumsq_ref[...] = acc_q[...]


_tc_stats = pl.pallas_call(
    _tc_stats_body,
    grid=(_GRID_A,),
    in_specs=[pl.BlockSpec((_TCB, _D1), lambda i: (i + _OFF_A, 0))],
    out_specs=[pl.BlockSpec((1, _D1), lambda i: (0, 0)),
               pl.BlockSpec((1, _D1), lambda i: (0, 0))],
    out_shape=[jax.ShapeDtypeStruct((1, _D1), jnp.float32),
               jax.ShapeDtypeStruct((1, _D1), jnp.float32)],
    scratch_shapes=[pltpu.VMEM((1, _D1), jnp.float32),
                    pltpu.VMEM((1, _D1), jnp.float32)],
)


def _encoder_body(parts_ref, tsum_ref, tsumsq_ref, x_ref, w_ref, b_ref,
                  out_ref):
    parts = parts_ref[:, 0, :]
    sums = (jnp.sum(parts[:, :_D1], axis=0, keepdims=True) + tsum_ref[...])
    sumsq = (jnp.sum(parts[:, _D1:], axis=0, keepdims=True) + tsumsq_ref[...])
    mean = sums / _N
    var = jnp.maximum((sumsq - sums * mean) / (_N - 1), 0.0)
    std = jnp.sqrt(var)
    new = (x_ref[...] - mean) / std
    new = jnp.where(std == 0.0, jnp.zeros_like(new), new)
    z = jnp.dot(new, w_ref[...], preferred_element_type=jnp.float32)
    out_ref[...] = jnp.tanh(z + b_ref[...])


_encoder = pl.pallas_call(
    _encoder_body,
    out_shape=jax.ShapeDtypeStruct((1, _D2), jnp.float32),
)


def _make_sc_dist(nch, s_rows):
    rw = _CH * nch

    @functools.partial(
        pl.kernel,
        out_type=jax.ShapeDtypeStruct((_NW, 1, _L), jnp.float32),
        mesh=_mesh,
        scratch_types=[
            pltpu.VMEM((_NB, _CH, _D2), jnp.float32),
            pltpu.VMEM((_D2,), jnp.float32),
            pltpu.VMEM((1, _L), jnp.float32),
            pltpu.SemaphoreType.DMA((_NB,)),
        ],
    )
    def sc_dist(mem_hbm, e_hbm, out_hbm, buf, e_v, min_v, sems):
        wid = lax.axis_index("s") * _NC + lax.axis_index("c")
        base = wid * rw

        pltpu.sync_copy(e_hbm.at[0], e_v)

        iota16 = lax.iota(jnp.int32, _L)
        dnums = lax.GatherDimensionNumbers(
            offset_dims=(), collapsed_slice_dims=(0,), start_index_map=(0,))
        perms = [(iota16 ^ k).reshape(_L, 1) for k in (1, 2, 4, 8)]

        def lane_total(v):
            for perm in perms:
                v = v + lax.gather(
                    v, perm, dnums, slice_sizes=(1,),
                    mode=lax.GatherScatterMode.PROMISE_IN_BOUNDS)
            return v

        def make_block_body(b):
            def body(i, m):
                r0 = _BLK * i
                ch = [[None, None] for _ in range(_BLK)]
                for c in range(_G2):
                    ev = e_v[pl.ds(c * _L, _L)]
                    for r in range(_BLK):
                        d = jnp.abs(buf[b, r0 + r, pl.ds(c * _L, _L)] - ev)
                        j = c % 2
                        ch[r][j] = d if ch[r][j] is None else ch[r][j] + d
                for r in range(_BLK):
                    m = jnp.minimum(m, lane_total(ch[r][0] + ch[r][1]))
                return m
            return body

        def make_row_body(b):
            def body(r, m):
                a0 = jnp.abs(buf[b, r, pl.ds(0, _L)] - e_v[pl.ds(0, _L)])
                a1 = jnp.abs(buf[b, r, pl.ds(_L, _L)] - e_v[pl.ds(_L, _L)])
                for c in range(2, _G2):
                    d = jnp.abs(buf[b, r, pl.ds(c * _L, _L)]
                                - e_v[pl.ds(c * _L, _L)])
                    if c % 2 == 0:
                        a0 = a0 + d
                    else:
                        a1 = a1 + d
                return jnp.minimum(m, lane_total(a0 + a1))
            return body

        def chunk_src(c):
            return mem_hbm.at[pl.ds(base + c * _CH, _CH)]

        for c in range(_NB - 1):
            pltpu.async_copy(chunk_src(c), buf.at[c], sems.at[c])

        def chunk_body(g, m):
            b = lax.rem(g, _NB)
            pltpu.make_async_copy(chunk_src(g), buf.at[b], sems.at[b]).wait()

            @pl.when(g + _NB - 1 < nch)
            def _():
                b2 = lax.rem(g + _NB - 1, _NB)
                pltpu.async_copy(chunk_src(g + _NB - 1), buf.at[b2],
                                 sems.at[b2])

            return lax.fori_loop(0, _CH // _BLK, make_block_body(b), m)

        m = jnp.full((_L,), jnp.inf, jnp.float32)
        m = lax.fori_loop(0, nch, chunk_body, m)

        tail_base, tail_rows, max_rows = _worker_tail(wid, nch, s_rows)
        if max_rows:
            @pl.when(tail_rows > 0)
            def _():
                pltpu.sync_copy(mem_hbm.at[pl.ds(tail_base, max_rows)],
                                buf.at[0, pl.ds(0, max_rows)])

            m = lax.fori_loop(0, tail_rows, make_row_body(0), m)

        min_v[...] = m.reshape(1, _L)
        pltpu.sync_copy(min_v, out_hbm.at[wid])

    return sc_dist


_sc_dist = _make_sc_dist(17, _S_C)


def _tc_dist_body(mem_ref, e_ref, out_ref, acc_m):
    i = pl.program_id(0)

    @pl.when(i == 0)
    def _():
        acc_m[...] = jnp.full_like(acc_m, jnp.inf)

    z = jnp.abs(mem_ref[...] - e_ref[...])
    ones = jnp.ones((_D2, 1), jnp.float32)
    d = jnp.dot(z, ones, preferred_element_type=jnp.float32)
    acc_m[...] = jnp.minimum(acc_m[...], jnp.min(d))

    @pl.when(i == _GRID_C - 1)
    def _():
        out_ref[...] = acc_m[...]


_tc_dist = pl.pallas_call(
    _tc_dist_body,
    grid=(_GRID_C,),
    in_specs=[pl.BlockSpec((_TCB, _D2), lambda i: (i + _OFF_C, 0)),
              pl.BlockSpec((1, _D2), lambda i: (0, 0))],
    out_specs=pl.BlockSpec((1, 1), lambda i: (0, 0)),
    out_shape=jax.ShapeDtypeStruct((1, 1), jnp.float32),
    scratch_shapes=[pltpu.VMEM((1, 1), jnp.float32)],
)


def kernel(x, memory, mem_data, W_enc, b_enc):
    parts_sc = _sc_stats(mem_data)
    tsum, tsumsq = _tc_stats(mem_data)
    e = _encoder(parts_sc, tsum, tsumsq, x, W_enc, b_enc.reshape(1, _D2))
    mins_sc = _sc_dist(memory, e)
    min_tc = _tc_dist(memory, e)
    return jnp.minimum(jnp.min(mins_sc), min_tc[0, 0])


# speedup vs baseline: 1.0968x; 1.0922x over previous
import functools

import jax
import jax.numpy as jnp
from jax import lax
from jax.experimental import pallas as pl
from jax.experimental.pallas import tpu as pltpu
from jax.experimental.pallas import tpu_sc as plsc

_N = 100000
_D1 = 256
_D2 = 512
_NC, _NS, _L = 2, 16, 16
_NW = _NC * _NS
_CH = 80
_NB = 3
_G1 = _D1 // _L
_G2 = _D2 // _L
_BLK = 8

_S_A = 40000
_S_C = 44000
_TCB = 4000
_GRID_A = (_N - _S_A) // _TCB
_GRID_C = (_N - _S_C) // _TCB
_OFF_A = _S_A // _TCB
_OFF_C = _S_C // _TCB

_mesh = plsc.VectorSubcoreMesh(
    core_axis_name="c", subcore_axis_name="s",
    num_cores=_NC, num_subcores=_NS)


def _worker_tail(wid, nch, s_rows):
    s_main = _NW * _CH * nch
    t = (s_rows - s_main) // 8
    q, r = divmod(t, _NW)
    n_units = q + jnp.where(wid < r, 1, 0)
    tail_base = s_main + 8 * (q * wid + jnp.minimum(wid, r))
    max_rows = 8 * (q + (1 if r else 0))
    return tail_base, 8 * n_units, max_rows


def _make_sc_stats(nch, s_rows):
    rw = _CH * nch

    @functools.partial(
        pl.kernel,
        out_type=jax.ShapeDtypeStruct((_NW, 1, 2 * _D1), jnp.float32),
        mesh=_mesh,
        scratch_types=[
            pltpu.VMEM((_NB, _CH, _D1), jnp.float32),
            pltpu.VMEM((1, 2 * _D1), jnp.float32),
            pltpu.SemaphoreType.DMA((_NB,)),
        ],
    )
    def sc_stats(md_hbm, out_hbm, buf, statbuf, sems):
        wid = lax.axis_index("s") * _NC + lax.axis_index("c")
        base = wid * rw

        def chunk_src(c):
            return md_hbm.at[pl.ds(base + c * _CH, _CH)]

        def make_rows2_body(b):
            def body(i, carry):
                out = list(carry)
                r0 = 2 * i
                for c in range(_G1):
                    v0 = buf[b, r0, pl.ds(c * _L, _L)]
                    v1 = buf[b, r0 + 1, pl.ds(c * _L, _L)]
                    out[c] = (out[c] + v0) + v1
                    out[_G1 + c] = (out[_G1 + c] + v0 * v0) + v1 * v1
                return tuple(out)
            return body

        def make_row_body(b):
            def body(r, carry):
                out = list(carry)
                for c in range(_G1):
                    v = buf[b, r, pl.ds(c * _L, _L)]
                    out[c] = out[c] + v
                    out[_G1 + c] = out[_G1 + c] + v * v
                return tuple(out)
            return body

        for c in range(_NB - 1):
            pltpu.async_copy(chunk_src(c), buf.at[c], sems.at[c])

        def chunk_body(g, carry):
            b = lax.rem(g, _NB)
            pltpu.make_async_copy(chunk_src(g), buf.at[b], sems.at[b]).wait()

            @pl.when(g + _NB - 1 < nch)
            def _():
                b2 = lax.rem(g + _NB - 1, _NB)
                pltpu.async_copy(chunk_src(g + _NB - 1), buf.at[b2],
                                 sems.at[b2])

            return lax.fori_loop(0, _CH // 2, make_rows2_body(b), carry)

        acc = tuple(jnp.zeros((_L,), jnp.float32) for _ in range(2 * _G1))
        acc = lax.fori_loop(0, nch, chunk_body, acc)

        tail_base, tail_rows, max_rows = _worker_tail(wid, nch, s_rows)
        if max_rows:
            @pl.when(tail_rows > 0)
            def _():
                pltpu.sync_copy(md_hbm.at[pl.ds(tail_base, max_rows)],
                                buf.at[0, pl.ds(0, max_rows)])

            acc = lax.fori_loop(0, tail_rows, make_row_body(0), acc)

        for c in range(_G1):
            statbuf[0, pl.ds(c * _L, _L)] = acc[c]
            statbuf[0, pl.ds(_D1 + c * _L, _L)] = acc[_G1 + c]
        pltpu.sync_copy(statbuf, out_hbm.at[wid])

    return sc_stats


_sc_stats = _make_sc_stats(15, _S_A)


def _tc_stats_body(md_ref, sums_ref, sumsq_ref, acc_s, acc_q):
    i = pl.program_id(0)

    @pl.when(i == 0)
    def _():
        acc_s[...] = jnp.zeros_like(acc_s)
        acc_q[...] = jnp.zeros_like(acc_q)

    x = md_ref[...]
    acc_s[...] += jnp.sum(x, axis=0, keepdims=True)
    acc_q[...] += jnp.sum(x * x, axis=0, keepdims=True)

    @pl.when(i == _GRID_A - 1)
    def _():
        sums_ref[...] = acc_s[...]
        sumsq_ref[...] = acc_q[...]


_tc_stats = pl.pallas_call(
    _tc_stats_body,
    grid=(_GRID_A,),
    in_specs=[pl.BlockSpec((_TCB, _D1), lambda i: (i + _OFF_A, 0))],
    out_specs=[pl.BlockSpec((1, _D1), lambda i: (0, 0)),
               pl.BlockSpec((1, _D1), lambda i: (0, 0))],
    out_shape=[jax.ShapeDtypeStruct((1, _D1), jnp.float32),
               jax.ShapeDtypeStruct((1, _D1), jnp.float32)],
    scratch_shapes=[pltpu.VMEM((1, _D1), jnp.float32),
                    pltpu.VMEM((1, _D1), jnp.float32)],
)


def _encoder_body(parts_ref, tsum_ref, tsumsq_ref, x_ref, w_ref, b_ref,
                  out_ref):
    parts = parts_ref[:, 0, :]
    sums = (jnp.sum(parts[:, :_D1], axis=0, keepdims=True) + tsum_ref[...])
    sumsq = (jnp.sum(parts[:, _D1:], axis=0, keepdims=True) + tsumsq_ref[...])
    mean = sums / _N
    var = jnp.maximum((sumsq - sums * mean) / (_N - 1), 0.0)
    std = jnp.sqrt(var)
    new = (x_ref[...] - mean) / std
    new = jnp.where(std == 0.0, jnp.zeros_like(new), new)
    z = jnp.dot(new, w_ref[...], preferred_element_type=jnp.float32)
    out_ref[...] = jnp.tanh(z + b_ref[...])


_encoder = pl.pallas_call(
    _encoder_body,
    out_shape=jax.ShapeDtypeStruct((1, _D2), jnp.float32),
)


def _make_sc_dist(nch, s_rows):
    rw = _CH * nch

    @functools.partial(
        pl.kernel,
        out_type=jax.ShapeDtypeStruct((_NW, 1, _L), jnp.float32),
        mesh=_mesh,
        scratch_types=[
            pltpu.VMEM((_NB, _CH, _D2), jnp.float32),
            pltpu.VMEM((_D2,), jnp.float32),
            pltpu.VMEM((1, _L), jnp.float32),
            pltpu.SemaphoreType.DMA((_NB,)),
        ],
    )
    def sc_dist(mem_hbm, e_hbm, out_hbm, buf, e_v, min_v, sems):
        wid = lax.axis_index("s") * _NC + lax.axis_index("c")
        base = wid * rw

        pltpu.sync_copy(e_hbm.at[0], e_v)

        iota16 = lax.iota(jnp.int32, _L)
        dnums = lax.GatherDimensionNumbers(
            offset_dims=(), collapsed_slice_dims=(0,), start_index_map=(0,))
        perms = [(iota16 ^ k).reshape(_L, 1) for k in (1, 2, 4, 8)]

        def lane_total(v):
            for perm in perms:
                v = v + lax.gather(
                    v, perm, dnums, slice_sizes=(1,),
                    mode=lax.GatherScatterMode.PROMISE_IN_BOUNDS)
            return v

        def make_block_body(b):
            def body(i, m):
                r0 = _BLK * i
                ch = [[None, None] for _ in range(_BLK)]
                for c in range(_G2):
                    ev = e_v[pl.ds(c * _L, _L)]
                    for r in range(_BLK):
                        d = jnp.abs(buf[b, r0 + r, pl.ds(c * _L, _L)] - ev)
                        j = c % 2
                        ch[r][j] = d if ch[r][j] is None else ch[r][j] + d
                for r in range(_BLK):
                    m = jnp.minimum(m, lane_total(ch[r][0] + ch[r][1]))
                return m
            return body

        def make_row_body(b):
            def body(r, m):
                a0 = jnp.abs(buf[b, r, pl.ds(0, _L)] - e_v[pl.ds(0, _L)])
                a1 = jnp.abs(buf[b, r, pl.ds(_L, _L)] - e_v[pl.ds(_L, _L)])
                for c in range(2, _G2):
                    d = jnp.abs(buf[b, r, pl.ds(c * _L, _L)]
                                - e_v[pl.ds(c * _L, _L)])
                    if c % 2 == 0:
                        a0 = a0 + d
                    else:
                        a1 = a1 + d
                return jnp.minimum(m, lane_total(a0 + a1))
            return body

        def chunk_src(c):
            return mem_hbm.at[pl.ds(base + c * _CH, _CH)]

        for c in range(_NB - 1):
            pltpu.async_copy(chunk_src(c), buf.at[c], sems.at[c])

        def chunk_body(g, m):
            b = lax.rem(g, _NB)
            pltpu.make_async_copy(chunk_src(g), buf.at[b], sems.at[b]).wait()

            @pl.when(g + _NB - 1 < nch)
            def _():
                b2 = lax.rem(g + _NB - 1, _NB)
                pltpu.async_copy(chunk_src(g + _NB - 1), buf.at[b2],
                                 sems.at[b2])

            return lax.fori_loop(0, _CH // _BLK, make_block_body(b), m)

        m = jnp.full((_L,), jnp.inf, jnp.float32)
        m = lax.fori_loop(0, nch, chunk_body, m)

        tail_base, tail_rows, max_rows = _worker_tail(wid, nch, s_rows)
        if max_rows:
            @pl.when(tail_rows > 0)
            def _():
                pltpu.sync_copy(mem_hbm.at[pl.ds(tail_base, max_rows)],
                                buf.at[0, pl.ds(0, max_rows)])

            m = lax.fori_loop(0, tail_rows, make_row_body(0), m)

        min_v[...] = m.reshape(1, _L)
        pltpu.sync_copy(min_v, out_hbm.at[wid])

    return sc_dist


_sc_dist = _make_sc_dist(17, _S_C)


def _tc_dist_body(mem_ref, e_ref, out_ref, acc_m):
    i = pl.program_id(0)

    @pl.when(i == 0)
    def _():
        acc_m[...] = jnp.full_like(acc_m, jnp.inf)

    z = jnp.abs(mem_ref[...] - e_ref[...])
    ones = jnp.ones((_D2, 1), jnp.float32)
    d = jnp.dot(z, ones, preferred_element_type=jnp.float32)
    acc_m[...] = jnp.minimum(acc_m[...], jnp.min(d))

    @pl.when(i == _GRID_C - 1)
    def _():
        out_ref[...] = acc_m[...]


_tc_dist = pl.pallas_call(
    _tc_dist_body,
    grid=(_GRID_C,),
    in_specs=[pl.BlockSpec((_TCB, _D2), lambda i: (i + _OFF_C, 0)),
              pl.BlockSpec((1, _D2), lambda i: (0, 0))],
    out_specs=pl.BlockSpec((1, 1), lambda i: (0, 0)),
    out_shape=jax.ShapeDtypeStruct((1, 1), jnp.float32),
    scratch_shapes=[pltpu.VMEM((1, 1), jnp.float32)],
)


def kernel(x, memory, mem_data, W_enc, b_enc):
    tsum, tsumsq = _tc_stats(mem_data)
    parts_sc = _sc_stats(mem_data)
    e = _encoder(parts_sc, tsum, tsumsq, x, W_enc, b_enc.reshape(1, _D2))
    min_tc = _tc_dist(memory, e)
    mins_sc = _sc_dist(memory, e)
    return jnp.minimum(jnp.min(mins_sc), min_tc[0, 0])
